# baseline (speedup 1.0000x reference)
import jax
import jax.numpy as jnp
from jax.experimental import pallas as pl

HEADS = 8
HD = 48
FOLD = 2
PW = 4
C = HEADS * HD
N = 1024
M = FOLD * FOLD * PW * PW

_BF = jnp.bfloat16
_F32 = jnp.float32
_HI = jax.lax.Precision.HIGHEST


def _bdot(a, b):
    return jnp.dot(a.astype(_BF), b.astype(_BF), preferred_element_type=_F32)


def _cluster_kernel(x_ref, wf_ref, bf_ref, wv_ref, bv_ref, wp_ref, bp_ref,
                    ab_ref, out_ref):
    xmat = x_ref[0].reshape(C, N)
    xf = _bdot(wf_ref[...], xmat) + bf_ref[...]
    val = _bdot(wv_ref[...], xmat) + bv_ref[...]

    ab = ab_ref[...]
    alpha = ab[:, 0:1]
    beta = ab[:, 1:2]

    n_iota = jax.lax.broadcasted_iota(jnp.int32, (N, M), 0)
    m_iota = jax.lax.broadcasted_iota(jnp.int32, (N, M), 1)
    w = n_iota // 32
    h = n_iota % 32
    quad = (w // 16) * 2 + (h // 16)
    m_of_n = quad * 16 + ((w % 16) // 4) * 4 + ((h % 16) // 4)
    pool = jnp.where(m_iota == m_of_n, 1.0 / 16.0, 0.0).astype(_F32)
    valid_nm = (m_iota // 16) == quad
    valid = jnp.transpose(valid_nm)
    ones_row = jnp.full((1, N), 1.0, dtype=_F32)

    cen_all = jnp.dot(xf, pool, preferred_element_type=_F32, precision=_HI)
    vc_all = _bdot(val, pool)

    xf3 = xf.reshape(HEADS, HD, N)
    xfn3 = xf3 / jnp.maximum(
        jnp.sqrt(jnp.sum(xf3 * xf3, axis=1, keepdims=True)), 1e-12)
    cen3 = cen_all.reshape(HEADS, HD, M)
    cenn3 = cen3 / jnp.maximum(
        jnp.sqrt(jnp.sum(cen3 * cen3, axis=1, keepdims=True)), 1e-12)

    outs = []
    for e in range(HEADS):
        v_h = jax.lax.slice(val, (e * HD, 0), ((e + 1) * HD, N))
        vc = jax.lax.slice(vc_all, (e * HD, 0), ((e + 1) * HD, M))
        xf_n = xfn3[e]
        cen_n = cenn3[e]

        sim = jax.nn.sigmoid(
            beta + alpha * jnp.einsum('cm,cn->mn',
                                      cen_n.astype(_BF), xf_n.astype(_BF),
                                      preferred_element_type=_F32))

        simv = jnp.where(valid, sim, -1.0)
        amax = jnp.max(simv, axis=0, keepdims=True)
        mi = jax.lax.broadcasted_iota(jnp.int32, (M, N), 0)
        first = jnp.min(jnp.where(simv >= amax, mi, M), axis=0, keepdims=True)
        simm = jnp.where(mi == first, sim, 0.0)

        v_aug = jnp.concatenate([v_h, ones_row], axis=0)
        agg_aug = jnp.einsum('cn,mn->cm', v_aug.astype(_BF), simm.astype(_BF),
                             preferred_element_type=_F32)
        agg = jax.lax.slice(agg_aug, (0, 0), (HD, M)) + vc
        denom = jax.lax.slice(agg_aug, (HD, 0), (HD + 1, M))
        out_m = agg / (denom + 1.0)
        out_h = _bdot(out_m, simm)
        outs.append(out_h)

    merged = jnp.concatenate(outs, axis=0)
    fin = _bdot(wp_ref[...], merged) + bp_ref[...]
    out_ref[0] = fin.reshape(C, 32, 32)


def kernel(x, Wf, bf, Wv, bv, Wp, bp, sim_alpha, sim_beta):
    B = x.shape[0]
    ab = jnp.concatenate([sim_alpha, sim_beta]).reshape(1, 2)
    bf2 = bf.reshape(C, 1)
    bv2 = bv.reshape(C, 1)
    bp2 = bp.reshape(C, 1)

    out = pl.pallas_call(
        _cluster_kernel,
        grid=(B,),
        in_specs=[
            pl.BlockSpec((1, C, 32, 32), lambda b: (b, 0, 0, 0)),
            pl.BlockSpec((C, C), lambda b: (0, 0)),
            pl.BlockSpec((C, 1), lambda b: (0, 0)),
            pl.BlockSpec((C, C), lambda b: (0, 0)),
            pl.BlockSpec((C, 1), lambda b: (0, 0)),
            pl.BlockSpec((C, C), lambda b: (0, 0)),
            pl.BlockSpec((C, 1), lambda b: (0, 0)),
            pl.BlockSpec((1, 2), lambda b: (0, 0)),
        ],
        out_specs=pl.BlockSpec((1, C, 32, 32), lambda b: (b, 0, 0, 0)),
        out_shape=jax.ShapeDtypeStruct((B, C, 32, 32), jnp.float32),
    )(x, Wf, bf2, Wv, bv2, Wp, bp2, ab)

    return out


# speedup vs baseline: 1.9196x; 1.9196x over previous
import numpy as np
import jax
import jax.numpy as jnp
from jax.experimental import pallas as pl

HEADS = 8
HD = 48
C = HEADS * HD
N = 1024
M = 64
MT = HEADS * M

_BF = jnp.bfloat16
_F32 = jnp.float32
_HI = jax.lax.Precision.HIGHEST


def _bdot(a, b):
    return jnp.dot(a.astype(_BF), b.astype(_BF), preferred_element_type=_F32)


def _constants():
    n = np.arange(N)
    w, h = n // 32, n % 32
    quad = (w // 16) * 2 + (h // 16)
    m_of_n = quad * 16 + ((w % 16) // 4) * 4 + ((h % 16) // 4)
    mm = np.arange(M)
    pool = ((mm[None, :] == m_of_n[:, None]) / 16.0).astype(np.float32)
    valid = (mm[:, None] // 16 == quad[None, :])
    valid_t = np.tile(valid, (HEADS, 1)).astype(np.float32)
    cc = np.arange(C)
    jj = np.arange(MT)
    bd = (cc[:, None] // HD == jj[None, :] // M).astype(np.float32)
    return jnp.asarray(pool), jnp.asarray(valid_t), jnp.asarray(bd)


def _cluster_kernel(x_ref, wf_ref, bf_ref, wv_ref, bv_ref, wp_ref, bp_ref,
                    ab_ref, pool_ref, valid_ref, bd_ref, out_ref):
    xmat = x_ref[0]
    xf = _bdot(wf_ref[...], xmat) + bf_ref[...]
    val = _bdot(wv_ref[...], xmat) + bv_ref[...]

    ab = ab_ref[...]
    alpha = ab[:, 0:1]
    beta = ab[:, 1:2]
    pool = pool_ref[...]
    valid = valid_ref[...]
    bd = bd_ref[...]

    cen_all = jnp.dot(xf, pool, preferred_element_type=_F32, precision=_HI)
    vc_all = _bdot(val, pool)

    xf3 = xf.reshape(HEADS, HD, N)
    xfn = (xf3 / jnp.maximum(
        jnp.sqrt(jnp.sum(xf3 * xf3, axis=1, keepdims=True)), 1e-12)
           ).reshape(C, N)
    cen3 = cen_all.reshape(HEADS, HD, M)
    cenn = (cen3 / jnp.maximum(
        jnp.sqrt(jnp.sum(cen3 * cen3, axis=1, keepdims=True)), 1e-12)
            ).reshape(C, M)

    cen_bd = jnp.tile(cenn, (1, HEADS)) * bd
    sim = jax.nn.sigmoid(
        beta + alpha * jnp.einsum('cm,cn->mn', cen_bd.astype(_BF),
                                  xfn.astype(_BF),
                                  preferred_element_type=_F32))

    simv = jnp.where(valid > 0.5, sim, -1.0).reshape(HEADS, M, N)
    amax = jnp.max(simv, axis=1, keepdims=True)
    mi = jax.lax.broadcasted_iota(jnp.int32, (1, M, N), 1)
    first = jnp.min(jnp.where(simv >= amax, mi, M), axis=1, keepdims=True)
    simm = (jnp.where(mi == first, sim.reshape(HEADS, M, N), 0.0)
            ).reshape(MT, N)

    v_aug = jnp.concatenate([val, jnp.full((1, N), 1.0, _F32)], axis=0)
    agg_aug = jnp.einsum('cn,mn->cm', v_aug.astype(_BF), simm.astype(_BF),
                         preferred_element_type=_F32)
    agg = jax.lax.slice(agg_aug, (0, 0), (C, MT))
    denom = jax.lax.slice(agg_aug, (C, 0), (C + 1, MT))

    vc_t = jnp.tile(vc_all, (1, HEADS))
    out_m = ((agg + vc_t) / (denom + 1.0)) * bd

    merged = _bdot(out_m, simm)
    out_ref[0] = _bdot(wp_ref[...], merged) + bp_ref[...]


def kernel(x, Wf, bf, Wv, bv, Wp, bp, sim_alpha, sim_beta):
    B = x.shape[0]
    x2 = x.reshape(B, C, N)
    ab = jnp.concatenate([sim_alpha, sim_beta]).reshape(1, 2)
    bf2 = bf.reshape(C, 1)
    bv2 = bv.reshape(C, 1)
    bp2 = bp.reshape(C, 1)
    pool, valid_t, bd = _constants()

    fixed = lambda b: (0, 0)
    out = pl.pallas_call(
        _cluster_kernel,
        grid=(B,),
        in_specs=[
            pl.BlockSpec((1, C, N), lambda b: (b, 0, 0)),
            pl.BlockSpec((C, C), fixed),
            pl.BlockSpec((C, 1), fixed),
            pl.BlockSpec((C, C), fixed),
            pl.BlockSpec((C, 1), fixed),
            pl.BlockSpec((C, C), fixed),
            pl.BlockSpec((C, 1), fixed),
            pl.BlockSpec((1, 2), fixed),
            pl.BlockSpec((N, M), fixed),
            pl.BlockSpec((MT, N), fixed),
            pl.BlockSpec((C, MT), fixed),
        ],
        out_specs=pl.BlockSpec((1, C, N), lambda b: (b, 0, 0)),
        out_shape=jax.ShapeDtypeStruct((B, C, N), jnp.float32),
    )(x2, Wf, bf2, Wv, bv2, Wp, bp2, ab, pool, valid_t, bd)

    return out.reshape(B, C, 32, 32)
